# baseline (speedup 1.0000x reference)
import functools

import jax
import jax.numpy as jnp
from jax import lax
from jax.experimental import pallas as pl
from jax.experimental.pallas import tpu as pltpu
from jax.experimental.pallas import tpu_sc as plsc

_LN_EPS = 1e-5
_TB = 128
_FC = 2048
_RANK_BLK = 512


def _router_body(xf_ref, rw_ref, rb_ref, s1_ref, s2_ref, w1s_ref, w2s_ref,
                 be_ref, nbu_ref, pres_ref):
    N, D = xf_ref.shape
    E = rw_ref.shape[1]
    NB = be_ref.shape[1]

    xf = xf_ref[...]
    logits = jnp.dot(xf, rw_ref[...],
                     preferred_element_type=jnp.float32) + rb_ref[...]
    m = jnp.max(logits, axis=-1, keepdims=True)
    ex = jnp.exp(logits - m)
    p = ex / jnp.sum(ex, axis=-1, keepdims=True)
    iota = lax.broadcasted_iota(jnp.int32, p.shape, 1)
    p1 = jnp.max(p, axis=-1, keepdims=True)
    i1 = jnp.min(jnp.where(p == p1, iota, E), axis=-1, keepdims=True)
    sel1 = iota == i1
    pm = jnp.where(sel1, -jnp.inf, p)
    p2 = jnp.max(pm, axis=-1, keepdims=True)
    i2 = jnp.min(jnp.where(pm == p2, iota, E), axis=-1, keepdims=True)
    sel2 = iota == i2
    w1s_ref[...] = p1
    w2s_ref[...] = p2
    pres_ref[...] = jnp.max((sel1 | sel2).astype(jnp.int32), axis=0,
                            keepdims=True)

    msk = (sel1 | sel2).astype(jnp.float32)
    r = lax.broadcasted_iota(jnp.int32, (_RANK_BLK, _RANK_BLK), 0)
    c = lax.broadcasted_iota(jnp.int32, (_RANK_BLK, _RANK_BLK), 1)
    ltri = (r > c).astype(jnp.bfloat16)
    carry = jnp.zeros((1, E), jnp.float32)
    ranks = []
    for b in range(N // _RANK_BLK):
        mb = msk[b * _RANK_BLK:(b + 1) * _RANK_BLK, :]
        ranks.append(jnp.dot(ltri, mb.astype(jnp.bfloat16),
                             preferred_element_type=jnp.float32) + carry)
        carry = carry + jnp.sum(mb, axis=0, keepdims=True)
    rank = jnp.concatenate(ranks, axis=0)
    counts = carry

    nblk = jnp.ceil(counts / float(_TB))
    eiota_r = lax.broadcasted_iota(jnp.int32, (E, E), 0)
    eiota_c = lax.broadcasted_iota(jnp.int32, (E, E), 1)
    sutri = (eiota_r < eiota_c).astype(jnp.float32)
    blk_off = jnp.dot(nblk, sutri,
                      preferred_element_type=jnp.float32)
    off = blk_off * float(_TB)
    nbu = blk_off[0, E - 1] + nblk[0, E - 1]
    nbu_ref[...] = jnp.full((1, 1), nbu, jnp.float32).astype(jnp.int32)

    slot = off + rank
    s1 = jnp.sum(jnp.where(sel1, slot, 0.0), axis=-1, keepdims=True)
    s2 = jnp.sum(jnp.where(sel2, slot, 0.0), axis=-1, keepdims=True)
    s1_ref[...] = s1.astype(jnp.int32)
    s2_ref[...] = s2.astype(jnp.int32)

    biota = lax.broadcasted_iota(jnp.int32, (E, NB), 1)
    ge = (biota >= jnp.transpose(blk_off).astype(jnp.int32)).astype(jnp.int32)
    be_ref[...] = jnp.sum(ge, axis=0, keepdims=True) - 1


def _run_router(xf, rw, rb, NB):
    N, D = xf.shape
    E = rw.shape[1]
    return pl.pallas_call(
        _router_body,
        in_specs=[
            pl.BlockSpec((N, D), lambda: (0, 0)),
            pl.BlockSpec((D, E), lambda: (0, 0)),
            pl.BlockSpec((1, E), lambda: (0, 0)),
        ],
        out_specs=[
            pl.BlockSpec((N, 1), lambda: (0, 0)),
            pl.BlockSpec((N, 1), lambda: (0, 0)),
            pl.BlockSpec((N, 1), lambda: (0, 0)),
            pl.BlockSpec((N, 1), lambda: (0, 0)),
            pl.BlockSpec((1, NB), lambda: (0, 0)),
            pl.BlockSpec((1, 1), lambda: (0, 0)),
            pl.BlockSpec((1, E), lambda: (0, 0)),
        ],
        out_shape=[
            jax.ShapeDtypeStruct((N, 1), jnp.int32),
            jax.ShapeDtypeStruct((N, 1), jnp.int32),
            jax.ShapeDtypeStruct((N, 1), jnp.float32),
            jax.ShapeDtypeStruct((N, 1), jnp.float32),
            jax.ShapeDtypeStruct((1, NB), jnp.int32),
            jax.ShapeDtypeStruct((1, 1), jnp.int32),
            jax.ShapeDtypeStruct((1, E), jnp.int32),
        ],
    )(xf, rw, rb)


_SC_CHUNK = 32


def _make_dispatch(N, D, NS):
    info = plsc.get_sparse_core_info()
    NW = info.num_cores * info.num_subcores
    tpw = N // NW
    C = _SC_CHUNK
    mesh = plsc.VectorSubcoreMesh(core_axis_name="c", subcore_axis_name="s")

    @functools.partial(
        pl.kernel, mesh=mesh,
        out_type=jax.ShapeDtypeStruct((NS, D), jnp.float32),
        scratch_types=[
            pltpu.VMEM((C,), jnp.int32),
            pltpu.VMEM((C,), jnp.int32),
            pltpu.VMEM((C, D), jnp.float32),
            pltpu.SemaphoreType.DMA,
            pltpu.SemaphoreType.DMA,
        ],
    )
    def dispatch(x_hbm, s1_hbm, s2_hbm, xs_hbm, i1_v, i2_v, rows_v, sa, sb):
        wid = lax.axis_index("s") * info.num_cores + lax.axis_index("c")
        base = wid * tpw
        for j in range(tpw // C):
            off = base + j * C
            pltpu.sync_copy(s1_hbm.at[pl.ds(off, C)], i1_v)
            pltpu.sync_copy(s2_hbm.at[pl.ds(off, C)], i2_v)
            pltpu.sync_copy(x_hbm.at[pl.ds(off, C)], rows_v)
            cp1 = pltpu.async_copy(rows_v, xs_hbm.at[i1_v], sa)
            cp2 = pltpu.async_copy(rows_v, xs_hbm.at[i2_v], sb)
            cp1.wait()
            cp2.wait()

    return dispatch


def _make_combine(N, D, NS):
    info = plsc.get_sparse_core_info()
    NW = info.num_cores * info.num_subcores
    tpw = N // NW
    C = _SC_CHUNK
    mesh = plsc.VectorSubcoreMesh(core_axis_name="c", subcore_axis_name="s")

    @functools.partial(
        pl.kernel, mesh=mesh,
        out_type=[jax.ShapeDtypeStruct((N, D), jnp.float32),
                  jax.ShapeDtypeStruct((N, D), jnp.float32)],
        scratch_types=[
            pltpu.VMEM((C,), jnp.int32),
            pltpu.VMEM((C,), jnp.int32),
            pltpu.VMEM((C, D), jnp.float32),
            pltpu.VMEM((C, D), jnp.float32),
            pltpu.SemaphoreType.DMA,
            pltpu.SemaphoreType.DMA,
        ],
    )
    def combine(os_hbm, s1_hbm, s2_hbm, g1_hbm, g2_hbm, i1_v, i2_v,
                r1_v, r2_v, sa, sb):
        wid = lax.axis_index("s") * info.num_cores + lax.axis_index("c")
        base = wid * tpw
        for j in range(tpw // C):
            off = base + j * C
            pltpu.sync_copy(s1_hbm.at[pl.ds(off, C)], i1_v)
            pltpu.sync_copy(s2_hbm.at[pl.ds(off, C)], i2_v)
            cp1 = pltpu.async_copy(os_hbm.at[i1_v], r1_v, sa)
            cp2 = pltpu.async_copy(os_hbm.at[i2_v], r2_v, sb)
            cp1.wait()
            cp2.wait()
            pltpu.sync_copy(r1_v, g1_hbm.at[pl.ds(off, C)])
            pltpu.sync_copy(r2_v, g2_hbm.at[pl.ds(off, C)])

    return combine


def _ffn_body(be_ref, nbu_ref, xs_ref, w1_ref, b1_ref, w2_ref, b2_ref,
              os_ref, acc_ref):
    f = pl.program_id(0)
    nf = pl.num_programs(0)
    b = pl.program_id(1)

    @pl.when(b < nbu_ref[0])
    def _():
        h = jnp.dot(xs_ref[...], w1_ref[0], preferred_element_type=jnp.float32)
        h = jnp.maximum(h + b1_ref[0], 0.0)
        part = jnp.dot(h, w2_ref[0], preferred_element_type=jnp.float32)
        sl = pl.ds(b * _TB, _TB)

        @pl.when(f == 0)
        def _():
            acc_ref[sl, :] = part.astype(jnp.bfloat16)

        @pl.when(f == nf - 1)
        def _():
            os_ref[...] = (acc_ref[sl, :].astype(jnp.float32) + part
                           + b2_ref[0])


def _run_ffn(xs, be, nbu, w1, b1, w2, b2):
    NS, D = xs.shape
    E, _, F = w1.shape
    NB = NS // _TB
    nf = F // _FC
    grid_spec = pltpu.PrefetchScalarGridSpec(
        num_scalar_prefetch=2,
        grid=(nf, NB),
        in_specs=[
            pl.BlockSpec((_TB, D), lambda f, b, be, nbu: (b, 0)),
            pl.BlockSpec((1, D, _FC), lambda f, b, be, nbu: (be[b], 0, f)),
            pl.BlockSpec((1, 1, _FC), lambda f, b, be, nbu: (be[b], 0, f)),
            pl.BlockSpec((1, _FC, D), lambda f, b, be, nbu: (be[b], f, 0)),
            pl.BlockSpec((1, 1, D), lambda f, b, be, nbu: (be[b], 0, 0)),
        ],
        out_specs=pl.BlockSpec(
            (_TB, D),
            lambda f, b, be, nbu: (jnp.where(f == nf - 1, b, 0), 0)),
        scratch_shapes=[pltpu.VMEM((NS, D), jnp.bfloat16)],
    )
    return pl.pallas_call(
        _ffn_body,
        grid_spec=grid_spec,
        out_shape=jax.ShapeDtypeStruct((NS, D), jnp.float32),
    )(be, nbu, xs, w1, b1.reshape(E, 1, F), w2, b2.reshape(E, 1, D))


def _ln_body(xf_ref, g1_ref, g2_ref, w1s_ref, w2s_ref, g_ref, be_ref, y_ref):
    y = (xf_ref[...] + g1_ref[...] * w1s_ref[...]
         + g2_ref[...] * w2s_ref[...])
    mu = jnp.mean(y, axis=-1, keepdims=True)
    d = y - mu
    var = jnp.mean(d * d, axis=-1, keepdims=True)
    y_ref[...] = d * lax.rsqrt(var + _LN_EPS) * g_ref[...] + be_ref[...]


def _run_ln(xf, g1, g2, w1s, w2s, gamma, beta):
    N, D = xf.shape
    T = 512
    return pl.pallas_call(
        _ln_body,
        grid=(N // T,),
        in_specs=[
            pl.BlockSpec((T, D), lambda t: (t, 0)),
            pl.BlockSpec((T, D), lambda t: (t, 0)),
            pl.BlockSpec((T, D), lambda t: (t, 0)),
            pl.BlockSpec((T, 1), lambda t: (t, 0)),
            pl.BlockSpec((T, 1), lambda t: (t, 0)),
            pl.BlockSpec((1, D), lambda t: (0, 0)),
            pl.BlockSpec((1, D), lambda t: (0, 0)),
        ],
        out_specs=pl.BlockSpec((T, D), lambda t: (t, 0)),
        out_shape=jax.ShapeDtypeStruct((N, D), jnp.float32),
    )(xf, g1, g2, w1s, w2s, gamma, beta)


def kernel(x, router_w, router_b, w1, b1, w2, b2, ln_gamma, ln_beta):
    B, S, D = x.shape
    E, _, F = w1.shape
    N = B * S
    NB = (N * 2) // _TB + E
    NS = NB * _TB

    xf = x.reshape(N, D)
    s1, s2, w1s, w2s, be, nbu, pres = _run_router(
        xf, router_w, router_b.reshape(1, E), NB)

    xs = _make_dispatch(N, D, NS)(xf, s1.reshape(N), s2.reshape(N))
    os_ = _run_ffn(xs, be.reshape(NB), nbu.reshape(1), w1, b1, w2, b2)
    g1, g2 = _make_combine(N, D, NS)(os_, s1.reshape(N), s2.reshape(N))

    y = _run_ln(xf, g1, g2, w1s, w2s, ln_gamma.reshape(1, D),
                ln_beta.reshape(1, D))

    present = pres[0] > 0
    vals = jnp.sort(jnp.where(present, jnp.arange(E, dtype=jnp.int32), E))
    sel = jnp.where(vals < E, vals, -1).astype(jnp.int32)
    return (y.reshape(B, S, D), sel)


# speedup vs baseline: 1.0556x; 1.0556x over previous
import functools

import jax
import jax.numpy as jnp
from jax import lax
from jax.experimental import pallas as pl
from jax.experimental.pallas import tpu as pltpu
from jax.experimental.pallas import tpu_sc as plsc

_LN_EPS = 1e-5
_TB = 256
_RANK_BLK = 512


def _router_body(xf_ref, rw_ref, rb_ref, s1_ref, s2_ref, w1s_ref, w2s_ref,
                 be_ref, nbu_ref, pres_ref):
    N, D = xf_ref.shape
    E = rw_ref.shape[1]
    NB = be_ref.shape[1]

    xf = xf_ref[...]
    logits = jnp.dot(xf, rw_ref[...],
                     preferred_element_type=jnp.float32) + rb_ref[...]
    m = jnp.max(logits, axis=-1, keepdims=True)
    ex = jnp.exp(logits - m)
    p = ex / jnp.sum(ex, axis=-1, keepdims=True)
    iota = lax.broadcasted_iota(jnp.int32, p.shape, 1)
    p1 = jnp.max(p, axis=-1, keepdims=True)
    i1 = jnp.min(jnp.where(p == p1, iota, E), axis=-1, keepdims=True)
    sel1 = iota == i1
    pm = jnp.where(sel1, -jnp.inf, p)
    p2 = jnp.max(pm, axis=-1, keepdims=True)
    i2 = jnp.min(jnp.where(pm == p2, iota, E), axis=-1, keepdims=True)
    sel2 = iota == i2
    w1s_ref[...] = p1
    w2s_ref[...] = p2
    pres_ref[...] = jnp.max((sel1 | sel2).astype(jnp.int32), axis=0,
                            keepdims=True)

    msk = (sel1 | sel2).astype(jnp.float32)
    r = lax.broadcasted_iota(jnp.int32, (_RANK_BLK, _RANK_BLK), 0)
    c = lax.broadcasted_iota(jnp.int32, (_RANK_BLK, _RANK_BLK), 1)
    ltri = (r > c).astype(jnp.bfloat16)
    carry = jnp.zeros((1, E), jnp.float32)
    ranks = []
    for b in range(N // _RANK_BLK):
        mb = msk[b * _RANK_BLK:(b + 1) * _RANK_BLK, :]
        ranks.append(jnp.dot(ltri, mb.astype(jnp.bfloat16),
                             preferred_element_type=jnp.float32) + carry)
        carry = carry + jnp.sum(mb, axis=0, keepdims=True)
    rank = jnp.concatenate(ranks, axis=0)
    counts = carry

    nblk = jnp.ceil(counts / float(_TB))
    eiota_r = lax.broadcasted_iota(jnp.int32, (E, E), 0)
    eiota_c = lax.broadcasted_iota(jnp.int32, (E, E), 1)
    sutri = (eiota_r < eiota_c).astype(jnp.float32)
    blk_off = jnp.dot(nblk, sutri,
                      preferred_element_type=jnp.float32)
    off = blk_off * float(_TB)
    nbu = blk_off[0, E - 1] + nblk[0, E - 1]
    nbu_ref[...] = jnp.full((1, 1), nbu, jnp.float32).astype(jnp.int32)

    slot = off + rank
    s1 = jnp.sum(jnp.where(sel1, slot, 0.0), axis=-1, keepdims=True)
    s2 = jnp.sum(jnp.where(sel2, slot, 0.0), axis=-1, keepdims=True)
    s1_ref[...] = s1.astype(jnp.int32)
    s2_ref[...] = s2.astype(jnp.int32)

    biota = lax.broadcasted_iota(jnp.int32, (E, NB), 1)
    ge = (biota >= jnp.transpose(blk_off).astype(jnp.int32)).astype(jnp.int32)
    be_ref[...] = jnp.sum(ge, axis=0, keepdims=True) - 1


def _run_router(xf, rw, rb, NB):
    N, D = xf.shape
    E = rw.shape[1]
    return pl.pallas_call(
        _router_body,
        in_specs=[
            pl.BlockSpec((N, D), lambda: (0, 0)),
            pl.BlockSpec((D, E), lambda: (0, 0)),
            pl.BlockSpec((1, E), lambda: (0, 0)),
        ],
        out_specs=[
            pl.BlockSpec((N, 1), lambda: (0, 0)),
            pl.BlockSpec((N, 1), lambda: (0, 0)),
            pl.BlockSpec((N, 1), lambda: (0, 0)),
            pl.BlockSpec((N, 1), lambda: (0, 0)),
            pl.BlockSpec((1, NB), lambda: (0, 0)),
            pl.BlockSpec((1, 1), lambda: (0, 0)),
            pl.BlockSpec((1, E), lambda: (0, 0)),
        ],
        out_shape=[
            jax.ShapeDtypeStruct((N, 1), jnp.int32),
            jax.ShapeDtypeStruct((N, 1), jnp.int32),
            jax.ShapeDtypeStruct((N, 1), jnp.float32),
            jax.ShapeDtypeStruct((N, 1), jnp.float32),
            jax.ShapeDtypeStruct((1, NB), jnp.int32),
            jax.ShapeDtypeStruct((1, 1), jnp.int32),
            jax.ShapeDtypeStruct((1, E), jnp.int32),
        ],
    )(xf, rw, rb)


_SC_CHUNK = 32


def _make_dispatch(N, D, NS):
    info = plsc.get_sparse_core_info()
    NW = info.num_cores * info.num_subcores
    tpw = N // NW
    C = _SC_CHUNK
    mesh = plsc.VectorSubcoreMesh(core_axis_name="c", subcore_axis_name="s")

    @functools.partial(
        pl.kernel, mesh=mesh,
        out_type=jax.ShapeDtypeStruct((NS, D), jnp.float32),
        scratch_types=[
            pltpu.VMEM((C,), jnp.int32),
            pltpu.VMEM((C,), jnp.int32),
            pltpu.VMEM((C, D), jnp.float32),
            pltpu.SemaphoreType.DMA,
            pltpu.SemaphoreType.DMA,
        ],
    )
    def dispatch(x_hbm, s1_hbm, s2_hbm, xs_hbm, i1_v, i2_v, rows_v, sa, sb):
        wid = lax.axis_index("s") * info.num_cores + lax.axis_index("c")
        base = wid * tpw
        for j in range(tpw // C):
            off = base + j * C
            pltpu.sync_copy(s1_hbm.at[pl.ds(off, C)], i1_v)
            pltpu.sync_copy(s2_hbm.at[pl.ds(off, C)], i2_v)
            pltpu.sync_copy(x_hbm.at[pl.ds(off, C)], rows_v)
            cp1 = pltpu.async_copy(rows_v, xs_hbm.at[i1_v], sa)
            cp2 = pltpu.async_copy(rows_v, xs_hbm.at[i2_v], sb)
            cp1.wait()
            cp2.wait()

    return dispatch


def _make_combine(N, D, NS):
    info = plsc.get_sparse_core_info()
    NW = info.num_cores * info.num_subcores
    tpw = N // NW
    C = _SC_CHUNK
    mesh = plsc.VectorSubcoreMesh(core_axis_name="c", subcore_axis_name="s")

    @functools.partial(
        pl.kernel, mesh=mesh,
        out_type=[jax.ShapeDtypeStruct((N, D), jnp.float32),
                  jax.ShapeDtypeStruct((N, D), jnp.float32)],
        scratch_types=[
            pltpu.VMEM((C,), jnp.int32),
            pltpu.VMEM((C,), jnp.int32),
            pltpu.VMEM((C, D), jnp.float32),
            pltpu.VMEM((C, D), jnp.float32),
            pltpu.SemaphoreType.DMA,
            pltpu.SemaphoreType.DMA,
        ],
    )
    def combine(os_hbm, s1_hbm, s2_hbm, g1_hbm, g2_hbm, i1_v, i2_v,
                r1_v, r2_v, sa, sb):
        wid = lax.axis_index("s") * info.num_cores + lax.axis_index("c")
        base = wid * tpw
        for j in range(tpw // C):
            off = base + j * C
            pltpu.sync_copy(s1_hbm.at[pl.ds(off, C)], i1_v)
            pltpu.sync_copy(s2_hbm.at[pl.ds(off, C)], i2_v)
            cp1 = pltpu.async_copy(os_hbm.at[i1_v], r1_v, sa)
            cp2 = pltpu.async_copy(os_hbm.at[i2_v], r2_v, sb)
            cp1.wait()
            cp2.wait()
            pltpu.sync_copy(r1_v, g1_hbm.at[pl.ds(off, C)])
            pltpu.sync_copy(r2_v, g2_hbm.at[pl.ds(off, C)])

    return combine


def _ffn1_body(be_ref, nbu_ref, xs_ref, w1_ref, b1_ref, h_ref):
    b = pl.program_id(0)

    @pl.when(b < nbu_ref[0])
    def _():
        h = jnp.dot(xs_ref[...], w1_ref[0], preferred_element_type=jnp.float32)
        h_ref[...] = jnp.maximum(h + b1_ref[0], 0.0).astype(jnp.bfloat16)


def _ffn2_body(be_ref, nbu_ref, h_ref, w2_ref, b2_ref, os_ref):
    b = pl.program_id(0)

    @pl.when(b < nbu_ref[0])
    def _():
        os_ref[...] = jnp.dot(h_ref[...].astype(jnp.float32), w2_ref[0],
                              preferred_element_type=jnp.float32) + b2_ref[0]


def _run_ffn(xs, be, nbu, w1, b1, w2, b2):
    NS, D = xs.shape
    E, _, F = w1.shape
    NB = NS // _TB
    grid1 = pltpu.PrefetchScalarGridSpec(
        num_scalar_prefetch=2,
        grid=(NB,),
        in_specs=[
            pl.BlockSpec((_TB, D), lambda b, be, nbu: (b, 0)),
            pl.BlockSpec((1, D, F), lambda b, be, nbu: (be[b], 0, 0)),
            pl.BlockSpec((1, 1, F), lambda b, be, nbu: (be[b], 0, 0)),
        ],
        out_specs=pl.BlockSpec((_TB, F), lambda b, be, nbu: (b, 0)),
    )
    h = pl.pallas_call(
        _ffn1_body,
        grid_spec=grid1,
        out_shape=jax.ShapeDtypeStruct((NS, F), jnp.bfloat16),
    )(be, nbu, xs, w1, b1.reshape(E, 1, F))
    grid2 = pltpu.PrefetchScalarGridSpec(
        num_scalar_prefetch=2,
        grid=(NB,),
        in_specs=[
            pl.BlockSpec((_TB, F), lambda b, be, nbu: (b, 0)),
            pl.BlockSpec((1, F, D), lambda b, be, nbu: (be[b], 0, 0)),
            pl.BlockSpec((1, 1, D), lambda b, be, nbu: (be[b], 0, 0)),
        ],
        out_specs=pl.BlockSpec((_TB, D), lambda b, be, nbu: (b, 0)),
    )
    return pl.pallas_call(
        _ffn2_body,
        grid_spec=grid2,
        out_shape=jax.ShapeDtypeStruct((NS, D), jnp.float32),
    )(be, nbu, h, w2, b2.reshape(E, 1, D))


def _ln_body(xf_ref, g1_ref, g2_ref, w1s_ref, w2s_ref, g_ref, be_ref, y_ref):
    y = (xf_ref[...] + g1_ref[...] * w1s_ref[...]
         + g2_ref[...] * w2s_ref[...])
    mu = jnp.mean(y, axis=-1, keepdims=True)
    d = y - mu
    var = jnp.mean(d * d, axis=-1, keepdims=True)
    y_ref[...] = d * lax.rsqrt(var + _LN_EPS) * g_ref[...] + be_ref[...]


def _run_ln(xf, g1, g2, w1s, w2s, gamma, beta):
    N, D = xf.shape
    T = 512
    return pl.pallas_call(
        _ln_body,
        grid=(N // T,),
        in_specs=[
            pl.BlockSpec((T, D), lambda t: (t, 0)),
            pl.BlockSpec((T, D), lambda t: (t, 0)),
            pl.BlockSpec((T, D), lambda t: (t, 0)),
            pl.BlockSpec((T, 1), lambda t: (t, 0)),
            pl.BlockSpec((T, 1), lambda t: (t, 0)),
            pl.BlockSpec((1, D), lambda t: (0, 0)),
            pl.BlockSpec((1, D), lambda t: (0, 0)),
        ],
        out_specs=pl.BlockSpec((T, D), lambda t: (t, 0)),
        out_shape=jax.ShapeDtypeStruct((N, D), jnp.float32),
    )(xf, g1, g2, w1s, w2s, gamma, beta)


def kernel(x, router_w, router_b, w1, b1, w2, b2, ln_gamma, ln_beta):
    B, S, D = x.shape
    E, _, F = w1.shape
    N = B * S
    NB = (N * 2) // _TB + E
    NS = NB * _TB

    xf = x.reshape(N, D)
    s1, s2, w1s, w2s, be, nbu, pres = _run_router(
        xf, router_w, router_b.reshape(1, E), NB)

    xs = _make_dispatch(N, D, NS)(xf, s1.reshape(N), s2.reshape(N))
    os_ = _run_ffn(xs, be.reshape(NB), nbu.reshape(1), w1, b1, w2, b2)
    g1, g2 = _make_combine(N, D, NS)(os_, s1.reshape(N), s2.reshape(N))

    y = _run_ln(xf, g1, g2, w1s, w2s, ln_gamma.reshape(1, D),
                ln_beta.reshape(1, D))

    present = pres[0] > 0
    vals = jnp.sort(jnp.where(present, jnp.arange(E, dtype=jnp.int32), E))
    sel = jnp.where(vals < E, vals, -1).astype(jnp.int32)
    return (y.reshape(B, S, D), sel)


# speedup vs baseline: 1.0656x; 1.0094x over previous
import functools

import jax
import jax.numpy as jnp
from jax import lax
from jax.experimental import pallas as pl
from jax.experimental.pallas import tpu as pltpu
from jax.experimental.pallas import tpu_sc as plsc

_LN_EPS = 1e-5
_TB = 256
_RANK_BLK = 512


def _router_body(xf_ref, rw_ref, rb_ref, s1_ref, s2_ref, w1s_ref, w2s_ref,
                 be_ref, nbu_ref, pres_ref):
    N, D = xf_ref.shape
    E = rw_ref.shape[1]
    NB = be_ref.shape[1]

    xf = xf_ref[...]
    logits = jnp.dot(xf, rw_ref[...],
                     preferred_element_type=jnp.float32) + rb_ref[...]
    m = jnp.max(logits, axis=-1, keepdims=True)
    ex = jnp.exp(logits - m)
    p = ex / jnp.sum(ex, axis=-1, keepdims=True)
    iota = lax.broadcasted_iota(jnp.int32, p.shape, 1)
    p1 = jnp.max(p, axis=-1, keepdims=True)
    i1 = jnp.min(jnp.where(p == p1, iota, E), axis=-1, keepdims=True)
    sel1 = iota == i1
    pm = jnp.where(sel1, -jnp.inf, p)
    p2 = jnp.max(pm, axis=-1, keepdims=True)
    i2 = jnp.min(jnp.where(pm == p2, iota, E), axis=-1, keepdims=True)
    sel2 = iota == i2
    w1s_ref[...] = p1
    w2s_ref[...] = p2
    pres_ref[...] = jnp.max((sel1 | sel2).astype(jnp.int32), axis=0,
                            keepdims=True)

    msk = (sel1 | sel2).astype(jnp.float32)
    r = lax.broadcasted_iota(jnp.int32, (_RANK_BLK, _RANK_BLK), 0)
    c = lax.broadcasted_iota(jnp.int32, (_RANK_BLK, _RANK_BLK), 1)
    ltri = (r > c).astype(jnp.bfloat16)
    carry = jnp.zeros((1, E), jnp.float32)
    ranks = []
    for b in range(N // _RANK_BLK):
        mb = msk[b * _RANK_BLK:(b + 1) * _RANK_BLK, :]
        ranks.append(jnp.dot(ltri, mb.astype(jnp.bfloat16),
                             preferred_element_type=jnp.float32) + carry)
        carry = carry + jnp.sum(mb, axis=0, keepdims=True)
    rank = jnp.concatenate(ranks, axis=0)
    counts = carry

    nblk = jnp.ceil(counts / float(_TB))
    eiota_r = lax.broadcasted_iota(jnp.int32, (E, E), 0)
    eiota_c = lax.broadcasted_iota(jnp.int32, (E, E), 1)
    sutri = (eiota_r < eiota_c).astype(jnp.float32)
    blk_off = jnp.dot(nblk, sutri,
                      preferred_element_type=jnp.float32)
    off = blk_off * float(_TB)
    nbu = blk_off[0, E - 1] + nblk[0, E - 1]
    nbu_ref[...] = jnp.full((1, 1), nbu, jnp.float32).astype(jnp.int32)

    slot = off + rank
    s1 = jnp.sum(jnp.where(sel1, slot, 0.0), axis=-1, keepdims=True)
    s2 = jnp.sum(jnp.where(sel2, slot, 0.0), axis=-1, keepdims=True)
    s1_ref[...] = s1.astype(jnp.int32)
    s2_ref[...] = s2.astype(jnp.int32)

    biota = lax.broadcasted_iota(jnp.int32, (E, NB), 1)
    ge = (biota >= jnp.transpose(blk_off).astype(jnp.int32)).astype(jnp.int32)
    be_ref[...] = jnp.sum(ge, axis=0, keepdims=True) - 1


def _run_router(xf, rw, rb, NB):
    N, D = xf.shape
    E = rw.shape[1]
    return pl.pallas_call(
        _router_body,
        in_specs=[
            pl.BlockSpec((N, D), lambda: (0, 0)),
            pl.BlockSpec((D, E), lambda: (0, 0)),
            pl.BlockSpec((1, E), lambda: (0, 0)),
        ],
        out_specs=[
            pl.BlockSpec((N, 1), lambda: (0, 0)),
            pl.BlockSpec((N, 1), lambda: (0, 0)),
            pl.BlockSpec((N, 1), lambda: (0, 0)),
            pl.BlockSpec((N, 1), lambda: (0, 0)),
            pl.BlockSpec((1, NB), lambda: (0, 0)),
            pl.BlockSpec((1, 1), lambda: (0, 0)),
            pl.BlockSpec((1, E), lambda: (0, 0)),
        ],
        out_shape=[
            jax.ShapeDtypeStruct((N, 1), jnp.int32),
            jax.ShapeDtypeStruct((N, 1), jnp.int32),
            jax.ShapeDtypeStruct((N, 1), jnp.float32),
            jax.ShapeDtypeStruct((N, 1), jnp.float32),
            jax.ShapeDtypeStruct((1, NB), jnp.int32),
            jax.ShapeDtypeStruct((1, 1), jnp.int32),
            jax.ShapeDtypeStruct((1, E), jnp.int32),
        ],
    )(xf, rw, rb)


_SC_CHUNK = 32


def _make_dispatch(N, D, NS):
    info = plsc.get_sparse_core_info()
    NW = info.num_cores * info.num_subcores
    tpw = N // NW
    C = _SC_CHUNK
    mesh = plsc.VectorSubcoreMesh(core_axis_name="c", subcore_axis_name="s")

    @functools.partial(
        pl.kernel, mesh=mesh,
        out_type=jax.ShapeDtypeStruct((NS, D), jnp.float32),
        scratch_types=[
            pltpu.VMEM((C,), jnp.int32),
            pltpu.VMEM((C,), jnp.int32),
            pltpu.VMEM((C, D), jnp.float32),
            pltpu.SemaphoreType.DMA,
            pltpu.SemaphoreType.DMA,
        ],
    )
    def dispatch(x_hbm, s1_hbm, s2_hbm, xs_hbm, i1_v, i2_v, rows_v, sa, sb):
        wid = lax.axis_index("s") * info.num_cores + lax.axis_index("c")
        base = wid * tpw
        for j in range(tpw // C):
            off = base + j * C
            pltpu.sync_copy(s1_hbm.at[pl.ds(off, C)], i1_v)
            pltpu.sync_copy(s2_hbm.at[pl.ds(off, C)], i2_v)
            pltpu.sync_copy(x_hbm.at[pl.ds(off, C)], rows_v)
            cp1 = pltpu.async_copy(rows_v, xs_hbm.at[i1_v], sa)
            cp2 = pltpu.async_copy(rows_v, xs_hbm.at[i2_v], sb)
            cp1.wait()
            cp2.wait()

    return dispatch


def _make_combine(N, D, NS):
    info = plsc.get_sparse_core_info()
    NW = info.num_cores * info.num_subcores
    tpw = N // NW
    C = _SC_CHUNK
    mesh = plsc.VectorSubcoreMesh(core_axis_name="c", subcore_axis_name="s")

    @functools.partial(
        pl.kernel, mesh=mesh,
        out_type=[jax.ShapeDtypeStruct((N, D), jnp.float32),
                  jax.ShapeDtypeStruct((N, D), jnp.float32)],
        scratch_types=[
            pltpu.VMEM((C,), jnp.int32),
            pltpu.VMEM((C,), jnp.int32),
            pltpu.VMEM((C, D), jnp.float32),
            pltpu.VMEM((C, D), jnp.float32),
            pltpu.SemaphoreType.DMA,
            pltpu.SemaphoreType.DMA,
        ],
    )
    def combine(os_hbm, s1_hbm, s2_hbm, g1_hbm, g2_hbm, i1_v, i2_v,
                r1_v, r2_v, sa, sb):
        wid = lax.axis_index("s") * info.num_cores + lax.axis_index("c")
        base = wid * tpw
        for j in range(tpw // C):
            off = base + j * C
            pltpu.sync_copy(s1_hbm.at[pl.ds(off, C)], i1_v)
            pltpu.sync_copy(s2_hbm.at[pl.ds(off, C)], i2_v)
            cp1 = pltpu.async_copy(os_hbm.at[i1_v], r1_v, sa)
            cp2 = pltpu.async_copy(os_hbm.at[i2_v], r2_v, sb)
            cp1.wait()
            cp2.wait()
            pltpu.sync_copy(r1_v, g1_hbm.at[pl.ds(off, C)])
            pltpu.sync_copy(r2_v, g2_hbm.at[pl.ds(off, C)])

    return combine


def _ffn_body(be_ref, nbu_ref, xs_ref, w1_ref, b1_ref, w2_ref, b2_ref,
              os_ref):
    b = pl.program_id(0)

    @pl.when(b < nbu_ref[0])
    def _():
        h = jnp.dot(xs_ref[...].astype(jnp.bfloat16), w1_ref[0],
                    preferred_element_type=jnp.float32)
        h = jnp.maximum(h + b1_ref[0], 0.0)
        os_ref[...] = jnp.dot(h, w2_ref[0],
                              preferred_element_type=jnp.float32) + b2_ref[0]


def _run_ffn(xs, be, nbu, w1, b1, w2, b2):
    NS, D = xs.shape
    E, _, F = w1.shape
    NB = NS // _TB
    grid_spec = pltpu.PrefetchScalarGridSpec(
        num_scalar_prefetch=2,
        grid=(NB,),
        in_specs=[
            pl.BlockSpec((_TB, D), lambda b, be, nbu: (b, 0)),
            pl.BlockSpec((1, D, F), lambda b, be, nbu: (be[b], 0, 0)),
            pl.BlockSpec((1, 1, F), lambda b, be, nbu: (be[b], 0, 0)),
            pl.BlockSpec((1, F, D), lambda b, be, nbu: (be[b], 0, 0)),
            pl.BlockSpec((1, 1, D), lambda b, be, nbu: (be[b], 0, 0)),
        ],
        out_specs=pl.BlockSpec((_TB, D), lambda b, be, nbu: (b, 0)),
    )
    return pl.pallas_call(
        _ffn_body,
        grid_spec=grid_spec,
        out_shape=jax.ShapeDtypeStruct((NS, D), jnp.float32),
    )(be, nbu, xs, w1.astype(jnp.bfloat16), b1.reshape(E, 1, F), w2,
      b2.reshape(E, 1, D))


def _ln_body(xf_ref, g1_ref, g2_ref, w1s_ref, w2s_ref, g_ref, be_ref, y_ref):
    y = (xf_ref[...] + g1_ref[...] * w1s_ref[...]
         + g2_ref[...] * w2s_ref[...])
    mu = jnp.mean(y, axis=-1, keepdims=True)
    d = y - mu
    var = jnp.mean(d * d, axis=-1, keepdims=True)
    y_ref[...] = d * lax.rsqrt(var + _LN_EPS) * g_ref[...] + be_ref[...]


def _run_ln(xf, g1, g2, w1s, w2s, gamma, beta):
    N, D = xf.shape
    T = 512
    return pl.pallas_call(
        _ln_body,
        grid=(N // T,),
        in_specs=[
            pl.BlockSpec((T, D), lambda t: (t, 0)),
            pl.BlockSpec((T, D), lambda t: (t, 0)),
            pl.BlockSpec((T, D), lambda t: (t, 0)),
            pl.BlockSpec((T, 1), lambda t: (t, 0)),
            pl.BlockSpec((T, 1), lambda t: (t, 0)),
            pl.BlockSpec((1, D), lambda t: (0, 0)),
            pl.BlockSpec((1, D), lambda t: (0, 0)),
        ],
        out_specs=pl.BlockSpec((T, D), lambda t: (t, 0)),
        out_shape=jax.ShapeDtypeStruct((N, D), jnp.float32),
    )(xf, g1, g2, w1s, w2s, gamma, beta)


def kernel(x, router_w, router_b, w1, b1, w2, b2, ln_gamma, ln_beta):
    B, S, D = x.shape
    E, _, F = w1.shape
    N = B * S
    NB = (N * 2) // _TB + E
    NS = NB * _TB

    xf = x.reshape(N, D)
    s1, s2, w1s, w2s, be, nbu, pres = _run_router(
        xf, router_w, router_b.reshape(1, E), NB)

    xs = _make_dispatch(N, D, NS)(xf, s1.reshape(N), s2.reshape(N))
    os_ = _run_ffn(xs, be.reshape(NB), nbu.reshape(1), w1, b1, w2, b2)
    g1, g2 = _make_combine(N, D, NS)(os_, s1.reshape(N), s2.reshape(N))

    y = _run_ln(xf, g1, g2, w1s, w2s, ln_gamma.reshape(1, D),
                ln_beta.reshape(1, D))

    present = pres[0] > 0
    vals = jnp.sort(jnp.where(present, jnp.arange(E, dtype=jnp.int32), E))
    sel = jnp.where(vals < E, vals, -1).astype(jnp.int32)
    return (y.reshape(B, S, D), sel)
